# baseline (speedup 1.0000x reference)
import functools

import jax
import jax.numpy as jnp
from jax import lax
from jax.experimental import pallas as pl
from jax.experimental.pallas import tpu as pltpu
from jax.experimental.pallas import tpu_sc as plsc

B = 8
PER = 4096
STRIDE = 4
K = 16
C_IN = 256
C_OUT = 512
M = PER // STRIDE
N = B * M


_MSUB = 512


def _topk_body(tpc_ref, prow_ref, idx_ref, rel_ref, *, boff):
    tpc = tpc_ref[...]
    prow = prow_ref[0]
    d2 = None
    for c in range(3):
        diff = tpc[:, c:c + 1] - prow[c:c + 1, :]
        d2 = diff * diff if d2 is None else d2 + diff * diff
    iota = lax.broadcasted_iota(jnp.int32, (_MSUB, PER), 1)
    inf = jnp.float32(jnp.inf)
    for k in range(K):
        idxv = jnp.argmin(d2, axis=1).reshape(_MSUB, 1)
        d2 = jnp.where(iota == idxv, inf, d2)
        idx_ref[:, k:k + 1] = idxv + boff
    selb = jnp.isinf(d2)
    for c in range(3):
        m = jnp.max(jnp.where(selb, prow[c:c + 1, :], -inf),
                    axis=1, keepdims=True)
        rel_ref[:, c:c + 1] = m - tpc[:, c:c + 1]


def _run_topk(tpcb, prowb, b):
    return pl.pallas_call(
        functools.partial(_topk_body, boff=b * PER),
        grid=(M // _MSUB,),
        in_specs=[
            pl.BlockSpec((_MSUB, 3), lambda s: (s, 0)),
            pl.BlockSpec((1, 3, PER), lambda s: (0, 0, 0)),
        ],
        out_specs=[
            pl.BlockSpec((_MSUB, K), lambda s: (s, 0)),
            pl.BlockSpec((_MSUB, 3), lambda s: (s, 0)),
        ],
        out_shape=[
            jax.ShapeDtypeStruct((M, K), jnp.int32),
            jax.ShapeDtypeStruct((M, 3), jnp.float32),
        ],
    )(tpcb, prowb)


_NW = 32
_TPW = M // _NW
_TPC = 8
_NCHUNK = _TPW // _TPC


def _sc_pool_body(x_hbm, idx_hbm, out_hbm, idx_v, rows_v, out_v):
    wid = lax.axis_index("s") * 2 + lax.axis_index("c")
    pltpu.sync_copy(idx_hbm.at[wid], idx_v)

    def chunk_body(cc, carry):
        pltpu.sync_copy(x_hbm.at[idx_v.at[cc]], rows_v)

        def tgt_body(t, carry2):
            for ch in range(C_IN // 16):
                sl = pl.ds(ch * 16, 16)
                acc = rows_v[t * K, sl]
                for r in range(1, K):
                    acc = jnp.maximum(acc, rows_v[t * K + r, sl])
                out_v[t, sl] = acc
            return carry2

        lax.fori_loop(0, _TPC, tgt_body, 0)
        pltpu.sync_copy(out_v, out_hbm.at[pl.ds(wid * _TPW + cc * _TPC, _TPC)])
        return carry

    lax.fori_loop(0, _NCHUNK, chunk_body, 0)


@functools.lru_cache(maxsize=1)
def _get_sc_pool():
    return pl.kernel(
        _sc_pool_body,
        out_type=jax.ShapeDtypeStruct((M, C_IN), jnp.float32),
        mesh=plsc.VectorSubcoreMesh(core_axis_name="c", subcore_axis_name="s"),
        scratch_types=[
            pltpu.VMEM((_NCHUNK, _TPC * K), jnp.int32),
            pltpu.VMEM((_TPC * K, C_IN), jnp.float32),
            pltpu.VMEM((_TPC, C_IN), jnp.float32),
        ],
    )


_RT = 512
_NT = N // _RT


def _mlp_body(rel_ref, feat_ref, w0_ref, w1_ref, h_ref, stats_ref):
    h = jnp.dot(feat_ref[...], w1_ref[...],
                preferred_element_type=jnp.float32)
    h = h + jnp.dot(rel_ref[...], w0_ref[...],
                    preferred_element_type=jnp.float32)
    h_ref[...] = h
    s1 = jnp.sum(h, axis=0, keepdims=True)
    s2 = jnp.sum(h * h, axis=0, keepdims=True)
    part = jnp.concatenate([s1, s2], axis=0)

    @pl.when(pl.program_id(0) == 0)
    def _init():
        stats_ref[...] = part

    @pl.when(pl.program_id(0) != 0)
    def _acc():
        stats_ref[...] += part


def _run_mlp(relp, feat, w0, w1):
    return pl.pallas_call(
        _mlp_body,
        grid=(_NT,),
        in_specs=[
            pl.BlockSpec((_RT, 8), lambda t: (t, 0)),
            pl.BlockSpec((_RT, C_IN), lambda t: (t, 0)),
            pl.BlockSpec((8, C_OUT), lambda t: (0, 0)),
            pl.BlockSpec((C_IN, C_OUT), lambda t: (0, 0)),
        ],
        out_specs=[
            pl.BlockSpec((_RT, C_OUT), lambda t: (t, 0)),
            pl.BlockSpec((2, C_OUT), lambda t: (0, 0)),
        ],
        out_shape=[
            jax.ShapeDtypeStruct((N, C_OUT), jnp.float32),
            jax.ShapeDtypeStruct((2, C_OUT), jnp.float32),
        ],
    )(relp, feat, w0, w1)


def _bn_body(h_ref, stats_ref, gamma_ref, beta_ref, out_ref):
    inv_n = jnp.float32(1.0 / N)
    mean = stats_ref[0:1, :] * inv_n
    var = stats_ref[1:2, :] * inv_n - mean * mean
    scale = gamma_ref[...] / jnp.sqrt(var + 1e-5)
    out_ref[...] = jnp.maximum((h_ref[...] - mean) * scale + beta_ref[...],
                               0.0)


def _run_bn(h, stats, gamma, beta):
    return pl.pallas_call(
        _bn_body,
        grid=(_NT,),
        in_specs=[
            pl.BlockSpec((_RT, C_OUT), lambda t: (t, 0)),
            pl.BlockSpec((2, C_OUT), lambda t: (0, 0)),
            pl.BlockSpec((1, C_OUT), lambda t: (0, 0)),
            pl.BlockSpec((1, C_OUT), lambda t: (0, 0)),
        ],
        out_specs=pl.BlockSpec((_RT, C_OUT), lambda t: (t, 0)),
        out_shape=jax.ShapeDtypeStruct((N, C_OUT), jnp.float32),
    )(h, stats, gamma, beta)


def kernel(p, x, o, W, gamma, beta):
    pb = p.reshape(B, PER, 3)
    tp = pb[:, ::STRIDE]
    tpc = tp.reshape(N, 3)
    prow = jnp.swapaxes(pb, 1, 2)

    feats, rels = [], []
    for b in range(B):
        idx_b, rel_b = _run_topk(tp[b], prow[b:b + 1], b)
        gidx_b = idx_b.reshape(_NW, _NCHUNK, _TPC * K)
        feats.append(_get_sc_pool()(x, gidx_b))
        rels.append(rel_b)
    feat = jnp.concatenate(feats, axis=0)
    rel = jnp.concatenate(rels, axis=0)

    relp = jnp.pad(rel, ((0, 0), (0, 5)))
    w0 = jnp.pad(W[:3], ((0, 5), (0, 0)))
    w1 = W[3:]
    h, stats = _run_mlp(relp, feat, w0, w1)
    out = _run_bn(h, stats, gamma.reshape(1, C_OUT), beta.reshape(1, C_OUT))

    return tpc, out, o // STRIDE


# speedup vs baseline: 1.0414x; 1.0414x over previous
import functools

import jax
import jax.numpy as jnp
from jax import lax
from jax.experimental import pallas as pl
from jax.experimental.pallas import tpu as pltpu
from jax.experimental.pallas import tpu_sc as plsc

B = 8
PER = 4096
STRIDE = 4
K = 16
C_IN = 256
C_OUT = 512
M = PER // STRIDE
N = B * M


_MSUB = 1024


def _topk_body(tpc_ref, prow_ref, idx_ref, rel_ref, *, boff):
    tpc = tpc_ref[...]
    prow = prow_ref[0]
    d2 = None
    for c in range(3):
        diff = tpc[:, c:c + 1] - prow[c:c + 1, :]
        d2 = diff * diff if d2 is None else d2 + diff * diff
    iota = lax.broadcasted_iota(jnp.int32, (_MSUB, PER), 1)
    inf = jnp.float32(jnp.inf)
    for k in range(K):
        idxv = jnp.argmin(d2, axis=1).reshape(_MSUB, 1)
        d2 = jnp.where(iota == idxv, inf, d2)
        idx_ref[:, k:k + 1] = idxv + boff
    selb = jnp.isinf(d2)
    for c in range(3):
        m = jnp.max(jnp.where(selb, prow[c:c + 1, :], -inf),
                    axis=1, keepdims=True)
        rel_ref[:, c:c + 1] = m - tpc[:, c:c + 1]


def _topk_body_g(tpc_ref, prow_ref, idx_ref, rel_ref):
    _topk_body(tpc_ref, prow_ref, idx_ref, rel_ref,
               boff=pl.program_id(0) * PER)


def _run_topk(tpc, prow):
    return pl.pallas_call(
        _topk_body_g,
        grid=(B, M // _MSUB),
        in_specs=[
            pl.BlockSpec((_MSUB, 3), lambda b, s: (b * (M // _MSUB) + s, 0)),
            pl.BlockSpec((1, 3, PER), lambda b, s: (b, 0, 0)),
        ],
        out_specs=[
            pl.BlockSpec((_MSUB, K), lambda b, s: (b * (M // _MSUB) + s, 0)),
            pl.BlockSpec((_MSUB, 3), lambda b, s: (b * (M // _MSUB) + s, 0)),
        ],
        out_shape=[
            jax.ShapeDtypeStruct((N, K), jnp.int32),
            jax.ShapeDtypeStruct((N, 3), jnp.float32),
        ],
    )(tpc, prow)


_NW = 32
_TPW = N // _NW
_TPC = 8
_NCHUNK = _TPW // _TPC


def _sc_pool_body(x_hbm, idx_hbm, out_hbm, idx_v, rows_v, out_v):
    wid = lax.axis_index("s") * 2 + lax.axis_index("c")
    pltpu.sync_copy(idx_hbm.at[wid], idx_v)

    def chunk_body(cc, carry):
        pltpu.sync_copy(x_hbm.at[idx_v.at[cc]], rows_v)

        def tgt_body(t, carry2):
            for ch in range(C_IN // 16):
                sl = pl.ds(ch * 16, 16)
                acc = rows_v[t * K, sl]
                for r in range(1, K):
                    acc = jnp.maximum(acc, rows_v[t * K + r, sl])
                out_v[t, sl] = acc
            return carry2

        lax.fori_loop(0, _TPC, tgt_body, 0)
        pltpu.sync_copy(out_v, out_hbm.at[pl.ds(wid * _TPW + cc * _TPC, _TPC)])
        return carry

    lax.fori_loop(0, _NCHUNK, chunk_body, 0)


@functools.lru_cache(maxsize=1)
def _get_sc_pool():
    return pl.kernel(
        _sc_pool_body,
        out_type=jax.ShapeDtypeStruct((N, C_IN), jnp.float32),
        mesh=plsc.VectorSubcoreMesh(core_axis_name="c", subcore_axis_name="s"),
        scratch_types=[
            pltpu.VMEM((_NCHUNK, _TPC * K), jnp.int32),
            pltpu.VMEM((_TPC * K, C_IN), jnp.float32),
            pltpu.VMEM((_TPC, C_IN), jnp.float32),
        ],
    )


_RT = 512
_NT = N // _RT


def _mlp_body(rel_ref, feat_ref, w0_ref, w1_ref, h_ref, stats_ref):
    h = jnp.dot(feat_ref[...], w1_ref[...],
                preferred_element_type=jnp.float32)
    h = h + jnp.dot(rel_ref[...], w0_ref[...],
                    preferred_element_type=jnp.float32)
    h_ref[...] = h
    s1 = jnp.sum(h, axis=0, keepdims=True)
    s2 = jnp.sum(h * h, axis=0, keepdims=True)
    part = jnp.concatenate([s1, s2], axis=0)

    @pl.when(pl.program_id(0) == 0)
    def _init():
        stats_ref[...] = part

    @pl.when(pl.program_id(0) != 0)
    def _acc():
        stats_ref[...] += part


def _run_mlp(relp, feat, w0, w1):
    return pl.pallas_call(
        _mlp_body,
        grid=(_NT,),
        in_specs=[
            pl.BlockSpec((_RT, 8), lambda t: (t, 0)),
            pl.BlockSpec((_RT, C_IN), lambda t: (t, 0)),
            pl.BlockSpec((8, C_OUT), lambda t: (0, 0)),
            pl.BlockSpec((C_IN, C_OUT), lambda t: (0, 0)),
        ],
        out_specs=[
            pl.BlockSpec((_RT, C_OUT), lambda t: (t, 0)),
            pl.BlockSpec((2, C_OUT), lambda t: (0, 0)),
        ],
        out_shape=[
            jax.ShapeDtypeStruct((N, C_OUT), jnp.float32),
            jax.ShapeDtypeStruct((2, C_OUT), jnp.float32),
        ],
    )(relp, feat, w0, w1)


def _bn_body(h_ref, stats_ref, gamma_ref, beta_ref, out_ref):
    inv_n = jnp.float32(1.0 / N)
    mean = stats_ref[0:1, :] * inv_n
    var = stats_ref[1:2, :] * inv_n - mean * mean
    scale = gamma_ref[...] / jnp.sqrt(var + 1e-5)
    out_ref[...] = jnp.maximum((h_ref[...] - mean) * scale + beta_ref[...],
                               0.0)


def _run_bn(h, stats, gamma, beta):
    return pl.pallas_call(
        _bn_body,
        grid=(_NT,),
        in_specs=[
            pl.BlockSpec((_RT, C_OUT), lambda t: (t, 0)),
            pl.BlockSpec((2, C_OUT), lambda t: (0, 0)),
            pl.BlockSpec((1, C_OUT), lambda t: (0, 0)),
            pl.BlockSpec((1, C_OUT), lambda t: (0, 0)),
        ],
        out_specs=pl.BlockSpec((_RT, C_OUT), lambda t: (t, 0)),
        out_shape=jax.ShapeDtypeStruct((N, C_OUT), jnp.float32),
    )(h, stats, gamma, beta)


def kernel(p, x, o, W, gamma, beta):
    pb = p.reshape(B, PER, 3)
    tp = pb[:, ::STRIDE]
    tpc = tp.reshape(N, 3)
    prow = jnp.swapaxes(pb, 1, 2)

    idx, rel = _run_topk(tpc, prow)
    gidx = idx.reshape(_NW, _NCHUNK, _TPC * K)
    feat = _get_sc_pool()(x, gidx)

    relp = jnp.pad(rel, ((0, 0), (0, 5)))
    w0 = jnp.pad(W[:3], ((0, 5), (0, 0)))
    w1 = W[3:]
    h, stats = _run_mlp(relp, feat, w0, w1)
    out = _run_bn(h, stats, gamma.reshape(1, C_OUT), beta.reshape(1, C_OUT))

    return tpc, out, o // STRIDE


# speedup vs baseline: 1.1359x; 1.0907x over previous
import functools

import jax
import jax.numpy as jnp
from jax import lax
from jax.experimental import pallas as pl
from jax.experimental.pallas import tpu as pltpu
from jax.experimental.pallas import tpu_sc as plsc

B = 8
PER = 4096
STRIDE = 4
K = 16
C_IN = 256
C_OUT = 512
M = PER // STRIDE
N = B * M


_MSUB = 512


def _topk_body(tpc_ref, prow_ref, idx_ref, rel_ref, *, boff):
    tpc = tpc_ref[...]
    prow = prow_ref[0]
    d2 = None
    for c in range(3):
        diff = tpc[:, c:c + 1] - prow[c:c + 1, :]
        d2 = diff * diff if d2 is None else d2 + diff * diff
    iota = lax.broadcasted_iota(jnp.int32, (_MSUB, PER), 1)
    inf = jnp.float32(jnp.inf)
    for k in range(K):
        idxv = jnp.argmin(d2, axis=1).reshape(_MSUB, 1)
        d2 = jnp.where(iota == idxv, inf, d2)
        idx_ref[:, k:k + 1] = idxv + boff
    selb = jnp.isinf(d2)
    for c in range(3):
        m = jnp.max(jnp.where(selb, prow[c:c + 1, :], -inf),
                    axis=1, keepdims=True)
        rel_ref[:, c:c + 1] = m - tpc[:, c:c + 1]


def _topk_body_g(tpc_ref, prow_ref, idx_ref, rel_ref):
    _topk_body(tpc_ref, prow_ref, idx_ref, rel_ref,
               boff=pl.program_id(0) * PER)


def _run_topk(tpc, prow):
    return pl.pallas_call(
        _topk_body_g,
        grid=(B, M // _MSUB),
        in_specs=[
            pl.BlockSpec((_MSUB, 3), lambda b, s: (b * (M // _MSUB) + s, 0)),
            pl.BlockSpec((1, 3, PER), lambda b, s: (b, 0, 0)),
        ],
        out_specs=[
            pl.BlockSpec((_MSUB, K), lambda b, s: (b * (M // _MSUB) + s, 0)),
            pl.BlockSpec((_MSUB, 3), lambda b, s: (b * (M // _MSUB) + s, 0)),
        ],
        out_shape=[
            jax.ShapeDtypeStruct((N, K), jnp.int32),
            jax.ShapeDtypeStruct((N, 3), jnp.float32),
        ],
    )(tpc, prow)


_NW = 32
_TPW = N // _NW
_TPC = 8
_NCHUNK = _TPW // _TPC


def _sc_pool_body(x_hbm, idx_hbm, out_hbm, idx_v, rows_v, out_v):
    wid = lax.axis_index("s") * 2 + lax.axis_index("c")
    pltpu.sync_copy(idx_hbm.at[wid], idx_v)

    def chunk_body(cc, carry):
        pltpu.sync_copy(x_hbm.at[idx_v.at[cc]], rows_v)

        def tgt_body(t, carry2):
            for ch in range(C_IN // 16):
                sl = pl.ds(ch * 16, 16)
                acc = rows_v[t * K, sl]
                for r in range(1, K):
                    acc = jnp.maximum(acc, rows_v[t * K + r, sl])
                out_v[t, sl] = acc
            return carry2

        lax.fori_loop(0, _TPC, tgt_body, 0)
        pltpu.sync_copy(out_v, out_hbm.at[pl.ds(wid * _TPW + cc * _TPC, _TPC)])
        return carry

    lax.fori_loop(0, _NCHUNK, chunk_body, 0)


@functools.lru_cache(maxsize=1)
def _get_sc_pool():
    return pl.kernel(
        _sc_pool_body,
        out_type=jax.ShapeDtypeStruct((N, C_IN), jnp.float32),
        mesh=plsc.VectorSubcoreMesh(core_axis_name="c", subcore_axis_name="s"),
        scratch_types=[
            pltpu.VMEM((_NCHUNK, _TPC * K), jnp.int32),
            pltpu.VMEM((_TPC * K, C_IN), jnp.float32),
            pltpu.VMEM((_TPC, C_IN), jnp.float32),
        ],
    )


_RT = 512
_NT = N // _RT


def _mlp_body(rel_ref, feat_ref, w0_ref, w1_ref, h_ref, stats_ref):
    h = jnp.dot(feat_ref[...], w1_ref[...],
                preferred_element_type=jnp.float32)
    h = h + jnp.dot(rel_ref[...], w0_ref[...],
                    preferred_element_type=jnp.float32)
    h_ref[...] = h
    s1 = jnp.sum(h, axis=0, keepdims=True)
    s2 = jnp.sum(h * h, axis=0, keepdims=True)
    part = jnp.concatenate([s1, s2], axis=0)

    @pl.when(pl.program_id(0) == 0)
    def _init():
        stats_ref[...] = part

    @pl.when(pl.program_id(0) != 0)
    def _acc():
        stats_ref[...] += part


def _run_mlp(relp, feat, w0, w1):
    return pl.pallas_call(
        _mlp_body,
        grid=(_NT,),
        in_specs=[
            pl.BlockSpec((_RT, 8), lambda t: (t, 0)),
            pl.BlockSpec((_RT, C_IN), lambda t: (t, 0)),
            pl.BlockSpec((8, C_OUT), lambda t: (0, 0)),
            pl.BlockSpec((C_IN, C_OUT), lambda t: (0, 0)),
        ],
        out_specs=[
            pl.BlockSpec((_RT, C_OUT), lambda t: (t, 0)),
            pl.BlockSpec((2, C_OUT), lambda t: (0, 0)),
        ],
        out_shape=[
            jax.ShapeDtypeStruct((N, C_OUT), jnp.float32),
            jax.ShapeDtypeStruct((2, C_OUT), jnp.float32),
        ],
    )(relp, feat, w0, w1)


def _bn_body(h_ref, stats_ref, gamma_ref, beta_ref, out_ref):
    inv_n = jnp.float32(1.0 / N)
    mean = stats_ref[0:1, :] * inv_n
    var = stats_ref[1:2, :] * inv_n - mean * mean
    scale = gamma_ref[...] / jnp.sqrt(var + 1e-5)
    out_ref[...] = jnp.maximum((h_ref[...] - mean) * scale + beta_ref[...],
                               0.0)


def _run_bn(h, stats, gamma, beta):
    return pl.pallas_call(
        _bn_body,
        grid=(_NT,),
        in_specs=[
            pl.BlockSpec((_RT, C_OUT), lambda t: (t, 0)),
            pl.BlockSpec((2, C_OUT), lambda t: (0, 0)),
            pl.BlockSpec((1, C_OUT), lambda t: (0, 0)),
            pl.BlockSpec((1, C_OUT), lambda t: (0, 0)),
        ],
        out_specs=pl.BlockSpec((_RT, C_OUT), lambda t: (t, 0)),
        out_shape=jax.ShapeDtypeStruct((N, C_OUT), jnp.float32),
    )(h, stats, gamma, beta)


def kernel(p, x, o, W, gamma, beta):
    pb = p.reshape(B, PER, 3)
    tp = pb[:, ::STRIDE]
    tpc = tp.reshape(N, 3)
    prow = jnp.swapaxes(pb, 1, 2)

    idx, rel = _run_topk(tpc, prow)
    gidx = idx.reshape(_NW, _NCHUNK, _TPC * K)
    feat = _get_sc_pool()(x, gidx)

    relp = jnp.pad(rel, ((0, 0), (0, 5)))
    w0 = jnp.pad(W[:3], ((0, 5), (0, 0)))
    w1 = W[3:]
    h, stats = _run_mlp(relp, feat, w0, w1)
    out = _run_bn(h, stats, gamma.reshape(1, C_OUT), beta.reshape(1, C_OUT))

    return tpc, out, o // STRIDE


# speedup vs baseline: 1.1627x; 1.0235x over previous
import functools

import jax
import jax.numpy as jnp
from jax import lax
from jax.experimental import pallas as pl
from jax.experimental.pallas import tpu as pltpu
from jax.experimental.pallas import tpu_sc as plsc

B = 8
PER = 4096
STRIDE = 4
K = 16
C_IN = 256
C_OUT = 512
M = PER // STRIDE
N = B * M


_MSUB = 256


def _topk_body(tpc_ref, prow_ref, idx_ref, rel_ref, *, boff):
    tpc = tpc_ref[...]
    prow = prow_ref[0]
    d2 = None
    for c in range(3):
        diff = tpc[:, c:c + 1] - prow[c:c + 1, :]
        d2 = diff * diff if d2 is None else d2 + diff * diff
    iota = lax.broadcasted_iota(jnp.int32, (_MSUB, PER), 1)
    inf = jnp.float32(jnp.inf)
    for k in range(K):
        idxv = jnp.argmin(d2, axis=1).reshape(_MSUB, 1)
        d2 = jnp.where(iota == idxv, inf, d2)
        idx_ref[:, k:k + 1] = idxv + boff
    selb = jnp.isinf(d2)
    for c in range(3):
        m = jnp.max(jnp.where(selb, prow[c:c + 1, :], -inf),
                    axis=1, keepdims=True)
        rel_ref[:, c:c + 1] = m - tpc[:, c:c + 1]


def _topk_body_g(tpc_ref, prow_ref, idx_ref, rel_ref):
    _topk_body(tpc_ref, prow_ref, idx_ref, rel_ref,
               boff=pl.program_id(0) * PER)


def _run_topk(tpc, prow):
    return pl.pallas_call(
        _topk_body_g,
        grid=(B, M // _MSUB),
        in_specs=[
            pl.BlockSpec((_MSUB, 3), lambda b, s: (b * (M // _MSUB) + s, 0)),
            pl.BlockSpec((1, 3, PER), lambda b, s: (b, 0, 0)),
        ],
        out_specs=[
            pl.BlockSpec((_MSUB, K), lambda b, s: (b * (M // _MSUB) + s, 0)),
            pl.BlockSpec((_MSUB, 3), lambda b, s: (b * (M // _MSUB) + s, 0)),
        ],
        out_shape=[
            jax.ShapeDtypeStruct((N, K), jnp.int32),
            jax.ShapeDtypeStruct((N, 3), jnp.float32),
        ],
    )(tpc, prow)


_NW = 32
_TPW = N // _NW
_TPC = 8
_NCHUNK = _TPW // _TPC


def _sc_pool_body(x_hbm, idx_hbm, out_hbm, idx_v, rows_v, out_v):
    wid = lax.axis_index("s") * 2 + lax.axis_index("c")
    pltpu.sync_copy(idx_hbm.at[wid], idx_v)

    def chunk_body(cc, carry):
        pltpu.sync_copy(x_hbm.at[idx_v.at[cc]], rows_v)

        def tgt_body(t, carry2):
            for ch in range(C_IN // 16):
                sl = pl.ds(ch * 16, 16)
                acc = rows_v[t * K, sl]
                for r in range(1, K):
                    acc = jnp.maximum(acc, rows_v[t * K + r, sl])
                out_v[t, sl] = acc
            return carry2

        lax.fori_loop(0, _TPC, tgt_body, 0)
        pltpu.sync_copy(out_v, out_hbm.at[pl.ds(wid * _TPW + cc * _TPC, _TPC)])
        return carry

    lax.fori_loop(0, _NCHUNK, chunk_body, 0)


@functools.lru_cache(maxsize=1)
def _get_sc_pool():
    return pl.kernel(
        _sc_pool_body,
        out_type=jax.ShapeDtypeStruct((N, C_IN), jnp.float32),
        mesh=plsc.VectorSubcoreMesh(core_axis_name="c", subcore_axis_name="s"),
        scratch_types=[
            pltpu.VMEM((_NCHUNK, _TPC * K), jnp.int32),
            pltpu.VMEM((_TPC * K, C_IN), jnp.float32),
            pltpu.VMEM((_TPC, C_IN), jnp.float32),
        ],
    )


_RT = 512
_NT = N // _RT


def _mlp_body(rel_ref, feat_ref, w0_ref, w1_ref, h_ref, stats_ref):
    h = jnp.dot(feat_ref[...], w1_ref[...],
                preferred_element_type=jnp.float32)
    h = h + jnp.dot(rel_ref[...], w0_ref[...],
                    preferred_element_type=jnp.float32)
    h_ref[...] = h
    s1 = jnp.sum(h, axis=0, keepdims=True)
    s2 = jnp.sum(h * h, axis=0, keepdims=True)
    part = jnp.concatenate([s1, s2], axis=0)

    @pl.when(pl.program_id(0) == 0)
    def _init():
        stats_ref[...] = part

    @pl.when(pl.program_id(0) != 0)
    def _acc():
        stats_ref[...] += part


def _run_mlp(relp, feat, w0, w1):
    return pl.pallas_call(
        _mlp_body,
        grid=(_NT,),
        in_specs=[
            pl.BlockSpec((_RT, 8), lambda t: (t, 0)),
            pl.BlockSpec((_RT, C_IN), lambda t: (t, 0)),
            pl.BlockSpec((8, C_OUT), lambda t: (0, 0)),
            pl.BlockSpec((C_IN, C_OUT), lambda t: (0, 0)),
        ],
        out_specs=[
            pl.BlockSpec((_RT, C_OUT), lambda t: (t, 0)),
            pl.BlockSpec((2, C_OUT), lambda t: (0, 0)),
        ],
        out_shape=[
            jax.ShapeDtypeStruct((N, C_OUT), jnp.float32),
            jax.ShapeDtypeStruct((2, C_OUT), jnp.float32),
        ],
    )(relp, feat, w0, w1)


def _bn_body(h_ref, stats_ref, gamma_ref, beta_ref, out_ref):
    inv_n = jnp.float32(1.0 / N)
    mean = stats_ref[0:1, :] * inv_n
    var = stats_ref[1:2, :] * inv_n - mean * mean
    scale = gamma_ref[...] / jnp.sqrt(var + 1e-5)
    out_ref[...] = jnp.maximum((h_ref[...] - mean) * scale + beta_ref[...],
                               0.0)


def _run_bn(h, stats, gamma, beta):
    return pl.pallas_call(
        _bn_body,
        grid=(_NT,),
        in_specs=[
            pl.BlockSpec((_RT, C_OUT), lambda t: (t, 0)),
            pl.BlockSpec((2, C_OUT), lambda t: (0, 0)),
            pl.BlockSpec((1, C_OUT), lambda t: (0, 0)),
            pl.BlockSpec((1, C_OUT), lambda t: (0, 0)),
        ],
        out_specs=pl.BlockSpec((_RT, C_OUT), lambda t: (t, 0)),
        out_shape=jax.ShapeDtypeStruct((N, C_OUT), jnp.float32),
    )(h, stats, gamma, beta)


def kernel(p, x, o, W, gamma, beta):
    pb = p.reshape(B, PER, 3)
    tp = pb[:, ::STRIDE]
    tpc = tp.reshape(N, 3)
    prow = jnp.swapaxes(pb, 1, 2)

    idx, rel = _run_topk(tpc, prow)
    gidx = idx.reshape(_NW, _NCHUNK, _TPC * K)
    feat = _get_sc_pool()(x, gidx)

    relp = jnp.pad(rel, ((0, 0), (0, 5)))
    w0 = jnp.pad(W[:3], ((0, 5), (0, 0)))
    w1 = W[3:]
    h, stats = _run_mlp(relp, feat, w0, w1)
    out = _run_bn(h, stats, gamma.reshape(1, C_OUT), beta.reshape(1, C_OUT))

    return tpc, out, o // STRIDE
